# Initial kernel scaffold; baseline (speedup 1.0000x reference)
import jax
import jax.numpy as jnp
from jax.experimental import pallas as pl


def kernel(embeddings, span_positions, span_nums, repeated_vector):
    raise NotImplementedError("write your pallas kernel here")


# speedup vs baseline: 1.7636x; 1.7636x over previous
import jax
import jax.numpy as jnp
from jax.experimental import pallas as pl
from jax.experimental.pallas import tpu as pltpu

_MAX_SPAN_LEN = 16
_WIN = 24


def _pool_body(starts_sm, ends_sm, sn_sm, emb_ref, out_ref, mask_ref, *, S, D):
    b = pl.program_id(0)
    sn = sn_sm[b]
    row_iota = jax.lax.broadcasted_iota(jnp.int32, (_WIN, D), 0)
    span_iota = jax.lax.broadcasted_iota(jnp.int32, (1, S), 1)
    mask_ref[0] = (span_iota < sn).astype(jnp.int32)
    for mi in range(S):
        s = starts_sm[b * S + mi]
        e = ends_sm[b * S + mi]
        base = pl.multiple_of((s >> 3) << 3, 8)
        win = emb_ref[0, pl.ds(base, _WIN), :]
        lo = s - base
        hi = jnp.minimum(e - base, lo + _MAX_SPAN_LEN)
        valid = (row_iota >= lo) & (row_iota < hi)
        summ = jnp.sum(jnp.where(valid, win, 0.0), axis=0)
        cnt = jnp.maximum(hi - lo, 1)
        scale = jnp.where(mi < sn, 1.0 / cnt.astype(jnp.float32), 0.0)
        out_ref[0, mi, :] = summ * scale


def kernel(embeddings, span_positions, span_nums, repeated_vector):
    B, L, D = embeddings.shape
    S = span_positions.shape[1]
    sp = span_positions.astype(jnp.int32) + 1
    starts = sp[..., 0].reshape(-1)
    ends = sp[..., 1].reshape(-1)
    sn = span_nums.astype(jnp.int32)

    import functools
    body = functools.partial(_pool_body, S=S, D=D)
    grid_spec = pltpu.PrefetchScalarGridSpec(
        num_scalar_prefetch=3,
        grid=(B,),
        in_specs=[pl.BlockSpec((1, L, D), lambda b, *_: (b, 0, 0))],
        out_specs=[pl.BlockSpec((1, S, D), lambda b, *_: (b, 0, 0)),
                   pl.BlockSpec((1, 1, S), lambda b, *_: (b, 0, 0))],
    )
    pooled, maski = pl.pallas_call(
        body,
        grid_spec=grid_spec,
        out_shape=[jax.ShapeDtypeStruct((B, S, D), jnp.float32),
                   jax.ShapeDtypeStruct((B, 1, S), jnp.int32)],
        compiler_params=pltpu.CompilerParams(
            dimension_semantics=("parallel",),
        ),
        name="span_mean_pool",
    )(starts, ends, sn, embeddings)
    return pooled, maski.reshape(B, S) > 0


# speedup vs baseline: 2.4959x; 1.4152x over previous
import functools

import jax
import jax.numpy as jnp
from jax.experimental import pallas as pl
from jax.experimental.pallas import tpu as pltpu

_MAX_SPAN_LEN = 16
_WIN = 24


def _pool_body(starts_sm, ends_sm, sn_sm, emb_hbm, out_ref, mask_ref,
               gbuf, sem, *, S, D):
    b = pl.program_id(0)
    nb = pl.num_programs(0)
    slot = jax.lax.rem(b, 2)
    nslot = 1 - slot

    def issue(bb, sl):
        for mi in range(S):
            s = starts_sm[bb * S + mi]
            base = pl.multiple_of((s >> 3) << 3, 8)
            pltpu.make_async_copy(
                emb_hbm.at[bb, pl.ds(base, _WIN), :],
                gbuf.at[sl, pl.ds(mi * _WIN, _WIN), :],
                sem.at[sl],
            ).start()

    @pl.when(b == 0)
    def _():
        issue(b, slot)

    @pl.when(b + 1 < nb)
    def _():
        issue(b + 1, nslot)

    sn = sn_sm[b]
    span_iota = jax.lax.broadcasted_iota(jnp.int32, (1, S), 1)
    mask_ref[0] = (span_iota < sn).astype(jnp.int32)

    pltpu.make_async_copy(
        emb_hbm.at[b, pl.ds(0, S * _WIN), :],
        gbuf.at[slot],
        sem.at[slot],
    ).wait()

    row_iota = jax.lax.broadcasted_iota(jnp.int32, (_WIN, D), 0)
    for mi in range(S):
        s = starts_sm[b * S + mi]
        e = ends_sm[b * S + mi]
        base = (s >> 3) << 3
        lo = s - base
        hi = jnp.minimum(e - base, lo + _MAX_SPAN_LEN)
        win = gbuf[slot, pl.ds(mi * _WIN, _WIN), :]
        valid = (row_iota >= lo) & (row_iota < hi)
        summ = jnp.sum(jnp.where(valid, win, 0.0), axis=0)
        cnt = jnp.maximum(hi - lo, 1)
        scale = jnp.where(mi < sn, 1.0 / cnt.astype(jnp.float32), 0.0)
        out_ref[0, mi, :] = summ * scale


def kernel(embeddings, span_positions, span_nums, repeated_vector):
    B, L, D = embeddings.shape
    S = span_positions.shape[1]
    sp = span_positions.astype(jnp.int32) + 1
    starts = sp[..., 0].reshape(-1)
    ends = sp[..., 1].reshape(-1)
    sn = span_nums.astype(jnp.int32)

    body = functools.partial(_pool_body, S=S, D=D)
    grid_spec = pltpu.PrefetchScalarGridSpec(
        num_scalar_prefetch=3,
        grid=(B,),
        in_specs=[pl.BlockSpec(memory_space=pl.ANY)],
        out_specs=[pl.BlockSpec((1, S, D), lambda b, *_: (b, 0, 0)),
                   pl.BlockSpec((1, 1, S), lambda b, *_: (b, 0, 0))],
        scratch_shapes=[
            pltpu.VMEM((2, S * _WIN, D), jnp.float32),
            pltpu.SemaphoreType.DMA((2,)),
        ],
    )
    pooled, maski = pl.pallas_call(
        body,
        grid_spec=grid_spec,
        out_shape=[jax.ShapeDtypeStruct((B, S, D), jnp.float32),
                   jax.ShapeDtypeStruct((B, 1, S), jnp.int32)],
        compiler_params=pltpu.CompilerParams(
            dimension_semantics=("arbitrary",),
        ),
        name="span_mean_pool_dma",
    )(starts, ends, sn, embeddings)
    return pooled, maski.reshape(B, S) > 0


# speedup vs baseline: 2.8301x; 1.1339x over previous
import functools

import jax
import jax.numpy as jnp
from jax.experimental import pallas as pl
from jax.experimental.pallas import tpu as pltpu

_MAX_SPAN_LEN = 16
_WIN = 24


def _pool_body(starts_sm, ends_sm, sn_sm, emb_hbm, sv_ref, ev_ref,
               out_ref, mask_ref, gbuf, sem, *, S, D):
    b = pl.program_id(0)
    nb = pl.num_programs(0)
    slot = jax.lax.rem(b, 2)
    nslot = 1 - slot

    def issue(bb, sl):
        for mi in range(S):
            s = starts_sm[bb * S + mi]
            base = pl.multiple_of((s >> 3) << 3, 8)
            pltpu.make_async_copy(
                emb_hbm.at[bb, pl.ds(base, _WIN), :],
                gbuf.at[sl, pl.ds(mi * _WIN, _WIN), :],
                sem.at[sl],
            ).start()

    @pl.when(b == 0)
    def _():
        issue(b, slot)

    @pl.when(b + 1 < nb)
    def _():
        issue(b + 1, nslot)

    sn = sn_sm[b]
    span_iota = jax.lax.broadcasted_iota(jnp.int32, (1, S), 1)
    valid_span = span_iota < sn
    mask_ref[0] = valid_span.astype(jnp.int32)

    sv = sv_ref[0]
    ev = ev_ref[0]
    lo = sv - ((sv >> 3) << 3)
    cnt = jnp.minimum(ev - sv, _MAX_SPAN_LEN)
    hi = lo + cnt
    inv = 1.0 / jnp.maximum(cnt, 1).astype(jnp.float32)
    scale = jnp.where(valid_span & (cnt > 0), inv, 0.0)

    k_iota = jax.lax.broadcasted_iota(jnp.int32, (S * _WIN, S), 0)
    mi_iota = jax.lax.broadcasted_iota(jnp.int32, (S * _WIN, S), 1)
    off = k_iota - mi_iota * _WIN
    wt = jnp.where((off >= lo) & (off < hi), scale, 0.0)

    pltpu.make_async_copy(
        emb_hbm.at[b, pl.ds(0, S * _WIN), :],
        gbuf.at[slot],
        sem.at[slot],
    ).wait()

    out_ref[0] = jax.lax.dot_general(
        wt, gbuf[slot], (((0,), (0,)), ((), ())),
        preferred_element_type=jnp.float32)


def kernel(embeddings, span_positions, span_nums, repeated_vector):
    B, L, D = embeddings.shape
    S = span_positions.shape[1]
    sp = span_positions.astype(jnp.int32) + 1
    starts = sp[..., 0].reshape(-1)
    ends = sp[..., 1].reshape(-1)
    sn = span_nums.astype(jnp.int32)
    sv = sp[..., 0].reshape(B, 1, S)
    ev = sp[..., 1].reshape(B, 1, S)

    body = functools.partial(_pool_body, S=S, D=D)
    grid_spec = pltpu.PrefetchScalarGridSpec(
        num_scalar_prefetch=3,
        grid=(B,),
        in_specs=[pl.BlockSpec(memory_space=pl.ANY),
                  pl.BlockSpec((1, 1, S), lambda b, *_: (b, 0, 0)),
                  pl.BlockSpec((1, 1, S), lambda b, *_: (b, 0, 0))],
        out_specs=[pl.BlockSpec((1, S, D), lambda b, *_: (b, 0, 0)),
                   pl.BlockSpec((1, 1, S), lambda b, *_: (b, 0, 0))],
        scratch_shapes=[
            pltpu.VMEM((2, S * _WIN, D), jnp.float32),
            pltpu.SemaphoreType.DMA((2,)),
        ],
    )
    pooled, maski = pl.pallas_call(
        body,
        grid_spec=grid_spec,
        out_shape=[jax.ShapeDtypeStruct((B, S, D), jnp.float32),
                   jax.ShapeDtypeStruct((B, 1, S), jnp.int32)],
        compiler_params=pltpu.CompilerParams(
            dimension_semantics=("arbitrary",),
        ),
        name="span_mean_pool_dma_mxu",
    )(starts, ends, sn, embeddings, sv, ev)
    return pooled, maski.reshape(B, S) > 0


# speedup vs baseline: 3.0494x; 1.0775x over previous
import functools

import jax
import jax.numpy as jnp
from jax.experimental import pallas as pl
from jax.experimental.pallas import tpu as pltpu

_MAX_SPAN_LEN = 16
_WIN = 24


def _pool_body(starts_sm, ends_sm, sn_sm, emb_hbm, sv_ref, ev_ref,
               out_ref, mask_ref, gbuf, sem16, sem8, *, S, D):
    b = pl.program_id(0)
    nb = pl.num_programs(0)
    slot = jax.lax.rem(b, 2)
    nslot = 1 - slot

    def span_bounds(bb, mi):
        s = starts_sm[bb * S + mi]
        e = ends_sm[bb * S + mi]
        base = pl.multiple_of((s >> 3) << 3, 8)
        spill = (s - base) + jnp.minimum(e - s, _MAX_SPAN_LEN) > 16
        return base, spill

    def issue(bb, sl):
        first_fill = bb <= 1
        for mi in range(S):
            base, spill = span_bounds(bb, mi)
            pltpu.make_async_copy(
                emb_hbm.at[bb, pl.ds(base, 16), :],
                gbuf.at[sl, pl.ds(mi * _WIN, 16), :],
                sem16.at[sl],
            ).start()

            @pl.when(spill | first_fill)
            def _():
                pltpu.make_async_copy(
                    emb_hbm.at[bb, pl.ds(base + 16, 8), :],
                    gbuf.at[sl, pl.ds(mi * _WIN + 16, 8), :],
                    sem8.at[sl],
                ).start()

    @pl.when(b == 0)
    def _():
        issue(b, slot)

    @pl.when(b + 1 < nb)
    def _():
        issue(b + 1, nslot)

    sn = sn_sm[b]
    span_iota = jax.lax.broadcasted_iota(jnp.int32, (1, S), 1)
    valid_span = span_iota < sn
    mask_ref[0] = valid_span.astype(jnp.int32)

    sv = sv_ref[0]
    ev = ev_ref[0]
    lo = sv - ((sv >> 3) << 3)
    cnt = jnp.minimum(ev - sv, _MAX_SPAN_LEN)
    hi = lo + cnt
    inv = 1.0 / jnp.maximum(cnt, 1).astype(jnp.float32)
    scale = jnp.where(valid_span & (cnt > 0), inv, 0.0)

    k_iota = jax.lax.broadcasted_iota(jnp.int32, (S * _WIN, S), 0)
    mi_iota = jax.lax.broadcasted_iota(jnp.int32, (S * _WIN, S), 1)
    off = k_iota - mi_iota * _WIN
    wt = jnp.where((off >= lo) & (off < hi), scale, 0.0)

    pltpu.make_async_copy(
        emb_hbm.at[b, pl.ds(0, S * 16), :],
        gbuf.at[slot, pl.ds(0, S * 16), :],
        sem16.at[slot],
    ).wait()
    first_fill = b <= 1
    for mi in range(S):
        _, spill = span_bounds(b, mi)

        @pl.when(spill | first_fill)
        def _():
            pltpu.make_async_copy(
                emb_hbm.at[b, pl.ds(0, 8), :],
                gbuf.at[slot, pl.ds(0, 8), :],
                sem8.at[slot],
            ).wait()

    out_ref[0] = jax.lax.dot_general(
        wt, gbuf[slot], (((0,), (0,)), ((), ())),
        preferred_element_type=jnp.float32)


def kernel(embeddings, span_positions, span_nums, repeated_vector):
    B, L, D = embeddings.shape
    S = span_positions.shape[1]
    sp = span_positions.astype(jnp.int32) + 1
    starts = sp[..., 0].reshape(-1)
    ends = sp[..., 1].reshape(-1)
    sn = span_nums.astype(jnp.int32)
    sv = sp[..., 0].reshape(B, 1, S)
    ev = sp[..., 1].reshape(B, 1, S)

    body = functools.partial(_pool_body, S=S, D=D)
    grid_spec = pltpu.PrefetchScalarGridSpec(
        num_scalar_prefetch=3,
        grid=(B,),
        in_specs=[pl.BlockSpec(memory_space=pl.ANY),
                  pl.BlockSpec((1, 1, S), lambda b, *_: (b, 0, 0)),
                  pl.BlockSpec((1, 1, S), lambda b, *_: (b, 0, 0))],
        out_specs=[pl.BlockSpec((1, S, D), lambda b, *_: (b, 0, 0)),
                   pl.BlockSpec((1, 1, S), lambda b, *_: (b, 0, 0))],
        scratch_shapes=[
            pltpu.VMEM((2, S * _WIN, D), jnp.float32),
            pltpu.SemaphoreType.DMA((2,)),
            pltpu.SemaphoreType.DMA((2,)),
        ],
    )
    pooled, maski = pl.pallas_call(
        body,
        grid_spec=grid_spec,
        out_shape=[jax.ShapeDtypeStruct((B, S, D), jnp.float32),
                   jax.ShapeDtypeStruct((B, 1, S), jnp.int32)],
        compiler_params=pltpu.CompilerParams(
            dimension_semantics=("arbitrary",),
        ),
        name="span_mean_pool_dma_mxu",
    )(starts, ends, sn, embeddings, sv, ev)
    return pooled, maski.reshape(B, S) > 0
